# Initial kernel scaffold; baseline (speedup 1.0000x reference)
import jax
import jax.numpy as jnp
from jax.experimental import pallas as pl


def kernel(x, edge_index, W1_self, W1_neigh, b1, W2_self, W2_neigh, b2):
    raise NotImplementedError("write your pallas kernel here")


# speedup vs baseline: 4.7389x; 4.7389x over previous
import functools

import jax
import jax.numpy as jnp
from jax import lax
from jax.experimental import pallas as pl
from jax.experimental.pallas import tpu as pltpu
from jax.experimental.pallas import tpu_sc as plsc

N = 10000
NPAD = 10240
D = 128
E = 320000
NC = 2
NS = 16
NW = NC * NS
CH = 64
NCH = 158
EPAD = NW * NCH * CH
DEGW = 16
ROWS_PER_SUB = NPAD // NS
ZCH = ROWS_PER_SUB // CH


def _agg_body(with_deg, *refs):
    if with_deg:
        (table, srci, dsti, zbuf_h, ones_h, zdeg_h,
         acc_out, deg_out,
         acc_sh, deg_sh, srcv, dstv, rows, onesv, zdegv, gsem) = refs
    else:
        (table, srci, dsti, zbuf_h,
         acc_out,
         acc_sh, srcv, dstv, rows, gsem) = refs

    c = lax.axis_index("c")
    s = lax.axis_index("s")
    wid = s * NC + c

    pltpu.sync_copy(zbuf_h, rows)
    if with_deg:
        pltpu.sync_copy(ones_h, onesv)
        pltpu.sync_copy(zdeg_h, zdegv)
    for k in range(ZCH):
        row0 = s * ROWS_PER_SUB + k * CH
        pltpu.sync_copy(rows, acc_sh.at[pl.ds(row0, CH)])
        if with_deg:
            pltpu.sync_copy(zdegv, deg_sh.at[pl.ds(row0, CH)])
    plsc.subcore_barrier()

    pltpu.sync_copy(srci.at[wid], srcv)
    pltpu.sync_copy(dsti.at[wid], dstv)

    def body(i, carry):
        pltpu.async_copy(table.at[srcv.at[i]], rows, gsem).wait()
        pltpu.sync_copy(rows, acc_sh.at[dstv.at[i]], add=True)
        if with_deg:
            pltpu.sync_copy(onesv, deg_sh.at[dstv.at[i]], add=True)
        return carry

    lax.fori_loop(0, NCH, body, 0)
    plsc.subcore_barrier()

    for k in range(ZCH):
        row0 = s * ROWS_PER_SUB + k * CH
        pltpu.sync_copy(acc_sh.at[pl.ds(row0, CH)], rows)
        pltpu.sync_copy(rows, acc_out.at[c, pl.ds(row0, CH)])
        if with_deg:
            pltpu.sync_copy(deg_sh.at[pl.ds(row0, CH)], zdegv)
            pltpu.sync_copy(zdegv, deg_out.at[c, pl.ds(row0, CH)])


def _make_agg(with_deg):
    mesh = plsc.VectorSubcoreMesh(core_axis_name="c", subcore_axis_name="s")
    out_type = [jax.ShapeDtypeStruct((NC, NPAD, D), jnp.float32)]
    scratch = [
        pltpu.VMEM_SHARED((NPAD, D), jnp.float32),
    ]
    if with_deg:
        out_type.append(jax.ShapeDtypeStruct((NC, NPAD, DEGW), jnp.float32))
        scratch.append(pltpu.VMEM_SHARED((NPAD, DEGW), jnp.float32))
    scratch += [
        pltpu.VMEM((NCH, CH), jnp.int32),
        pltpu.VMEM((NCH, CH), jnp.int32),
        pltpu.VMEM((CH, D), jnp.float32),
    ]
    if with_deg:
        scratch += [
            pltpu.VMEM((CH, DEGW), jnp.float32),
            pltpu.VMEM((CH, DEGW), jnp.float32),
        ]
    scratch.append(pltpu.SemaphoreType.DMA)
    return pl.kernel(
        functools.partial(_agg_body, with_deg),
        out_type=tuple(out_type),
        mesh=mesh,
        scratch_types=tuple(scratch),
        compiler_params=pltpu.CompilerParams(use_tc_tiling_on_sc=False),
    )


ROWT = 512
GRID = NPAD // ROWT


def _tc1_body(x_ref, a0_ref, a1_ref, d0_ref, d1_ref,
              w1s_ref, w1n_ref, b1_ref, w2s_ref, w2n_ref,
              s2_ref, p2_ref):
    deg = d0_ref[:, :1] + d1_ref[:, :1]
    inv = 1.0 / jnp.maximum(deg, 1.0)
    agg = (a0_ref[...] + a1_ref[...]) * inv
    h = jnp.dot(x_ref[...], w1s_ref[...], preferred_element_type=jnp.float32)
    h = h + jnp.dot(agg, w1n_ref[...], preferred_element_type=jnp.float32)
    h = jnp.maximum(h + b1_ref[...], 0.0)
    s2_ref[...] = jnp.dot(h, w2s_ref[...], preferred_element_type=jnp.float32)
    p2_ref[...] = jnp.dot(h, w2n_ref[...], preferred_element_type=jnp.float32)


def _tc2_body(s2_ref, a0_ref, a1_ref, d0_ref, d1_ref, b2_ref, o_ref):
    deg = d0_ref[:, :1] + d1_ref[:, :1]
    inv = 1.0 / jnp.maximum(deg, 1.0)
    agg = (a0_ref[...] + a1_ref[...]) * inv
    o_ref[...] = jnp.maximum(s2_ref[...] + agg + b2_ref[...], 0.0)


def _row_spec(w):
    return pl.BlockSpec((ROWT, w), lambda i: (i, 0))


def _full_spec(shape):
    return pl.BlockSpec(shape, lambda i: tuple(0 for _ in shape))


_tc1 = pl.pallas_call(
    _tc1_body,
    grid=(GRID,),
    in_specs=[
        _row_spec(D), _row_spec(D), _row_spec(D),
        _row_spec(DEGW), _row_spec(DEGW),
        _full_spec((D, 2 * D)), _full_spec((D, 2 * D)), _full_spec((1, 2 * D)),
        _full_spec((2 * D, D)), _full_spec((2 * D, D)),
    ],
    out_specs=[_row_spec(D), _row_spec(D)],
    out_shape=[
        jax.ShapeDtypeStruct((NPAD, D), jnp.float32),
        jax.ShapeDtypeStruct((NPAD, D), jnp.float32),
    ],
)

_tc2 = pl.pallas_call(
    _tc2_body,
    grid=(GRID,),
    in_specs=[
        _row_spec(D), _row_spec(D), _row_spec(D),
        _row_spec(DEGW), _row_spec(DEGW),
        _full_spec((1, D)),
    ],
    out_specs=_row_spec(D),
    out_shape=jax.ShapeDtypeStruct((NPAD, D), jnp.float32),
)


def kernel(x, edge_index, W1_self, W1_neigh, b1, W2_self, W2_neigh, b2):
    src = edge_index[0].astype(jnp.int32)
    dst = edge_index[1].astype(jnp.int32)
    pad = EPAD - E
    src_p = jnp.concatenate(
        [src, jnp.zeros((pad,), jnp.int32)]).reshape(NW, NCH, CH)
    dst_p = jnp.concatenate(
        [dst, jnp.full((pad,), N, jnp.int32)]).reshape(NW, NCH, CH)
    x_p = jnp.pad(x, ((0, NPAD - N), (0, 0)))

    zrow = jnp.zeros((CH, D), jnp.float32)
    ones_c = jnp.zeros((CH, DEGW), jnp.float32).at[:, 0].set(1.0)
    zdeg = jnp.zeros((CH, DEGW), jnp.float32)

    agg_deg = _make_agg(True)
    agg_only = _make_agg(False)

    accA, degt = agg_deg(x_p, src_p, dst_p, zrow, ones_c, zdeg)
    s2, p2 = _tc1(x_p, accA[0], accA[1], degt[0], degt[1],
                  W1_self, W1_neigh, b1.reshape(1, -1), W2_self, W2_neigh)
    accB, = agg_only(p2, src_p, dst_p, zrow)
    out = _tc2(s2, accB[0], accB[1], degt[0], degt[1], b2.reshape(1, -1))
    return out[:N]
